# baseline (speedup 1.0000x reference)
import functools

import jax
import jax.numpy as jnp
from jax import lax
from jax.experimental import pallas as pl
from jax.experimental.pallas import tpu as pltpu
from jax.experimental.pallas import tpu_sc as plsc

NN = 10000
NP = 10240
NE = 320000
CHUNK = 128
TPC = 16
CPT = 160
NEP = TPC * CPT * CHUNK
RPT = NP // TPC
PAD_COL = NN + 8
NG = 64
D = 128

_f32 = jnp.float32
_mesh = plsc.VectorSubcoreMesh(core_axis_name="c", subcore_axis_name="s")


DEGW = 128


def _deg_kernel(zeros_hbm, ones_hbm, cols_hbm, out_hbm, ones_v, col_v, hist):
    c = lax.axis_index("c")
    s = lax.axis_index("s")
    w = c * TPC + s

    pltpu.sync_copy(ones_hbm, ones_v)
    pltpu.sync_copy(zeros_hbm.at[pl.ds(s * RPT, RPT)],
                    hist.at[pl.ds(s * RPT, RPT)])
    plsc.subcore_barrier()

    @pl.loop(0, CPT // IB)
    def _(blk):
        pltpu.sync_copy(cols_hbm.at[w, pl.ds(blk * IB, IB)], col_v)

        @pl.loop(0, IB)
        def _(g):
            pltpu.sync_copy(ones_v, hist.at[col_v.at[g]], add=True)

    plsc.subcore_barrier()
    pltpu.sync_copy(hist.at[pl.ds(s * RPT, RPT)],
                    out_hbm.at[c, pl.ds(s * RPT, RPT)])


def _deg_call(cols):
    zeros = jnp.zeros((NP, DEGW), _f32)
    ones = jnp.ones((CHUNK, DEGW), _f32)
    k = pl.kernel(
        _deg_kernel,
        out_type=jax.ShapeDtypeStruct((2, NP, DEGW), _f32),
        mesh=_mesh,
        scratch_types=[pltpu.VMEM((CHUNK, DEGW), _f32),
                       pltpu.VMEM((IB, CHUNK), jnp.int32),
                       pltpu.VMEM_SHARED((NP, DEGW), _f32)],
    )
    return k(zeros, ones, cols)


IB = 40


def _agg_kernel(table_hbm, rows_hbm, cols_hbm, out_hbm, row_v, col_v, gb0,
                gb1, sem0, sem1, ssem0, ssem1, acc):
    c = lax.axis_index("c")
    s = lax.axis_index("s")
    w = c * TPC + s

    pltpu.sync_copy(table_hbm.at[pl.ds(c * NP + s * RPT, RPT)],
                    acc.at[pl.ds(s * RPT, RPT)])
    plsc.subcore_barrier()

    @pl.loop(0, CPT // IB)
    def _(blk):
        pltpu.sync_copy(rows_hbm.at[w, pl.ds(blk * IB, IB)], row_v)
        pltpu.sync_copy(cols_hbm.at[w, pl.ds(blk * IB, IB)], col_v)
        pltpu.async_copy(table_hbm.at[row_v.at[0]], gb0, sem0)
        pltpu.async_copy(table_hbm.at[row_v.at[1]], gb1, sem1)

        @pl.loop(0, IB, step=2)
        def _(g):
            pltpu.make_async_copy(table_hbm.at[row_v.at[g]], gb0, sem0).wait()
            pltpu.async_copy(gb0, acc.at[col_v.at[g]], ssem0, add=True)
            pltpu.make_async_copy(table_hbm.at[row_v.at[g + 1]], gb1,
                                  sem1).wait()
            pltpu.async_copy(gb1, acc.at[col_v.at[g + 1]], ssem1, add=True)

            @pl.when(g + 2 < IB)
            def _():
                pltpu.make_async_copy(gb0, acc.at[col_v.at[g]], ssem0).wait()
                pltpu.async_copy(table_hbm.at[row_v.at[g + 2]], gb0, sem0)
                pltpu.make_async_copy(gb1, acc.at[col_v.at[g + 1]],
                                      ssem1).wait()
                pltpu.async_copy(table_hbm.at[row_v.at[g + 3]], gb1, sem1)

        pltpu.make_async_copy(gb0, acc.at[col_v.at[IB - 2]], ssem0).wait()
        pltpu.make_async_copy(gb1, acc.at[col_v.at[IB - 1]], ssem1).wait()

    plsc.subcore_barrier()
    pltpu.sync_copy(acc.at[pl.ds(s * RPT, RPT)],
                    out_hbm.at[c, pl.ds(s * RPT, RPT)])


def _agg_call(table, rows, cols):
    k = pl.kernel(
        _agg_kernel,
        out_type=jax.ShapeDtypeStruct((2, NP, D), _f32),
        mesh=_mesh,
        scratch_types=[pltpu.VMEM((IB, CHUNK), jnp.int32),
                       pltpu.VMEM((IB, CHUNK), jnp.int32),
                       pltpu.VMEM((CHUNK, D), _f32),
                       pltpu.VMEM((CHUNK, D), _f32),
                       pltpu.SemaphoreType.DMA,
                       pltpu.SemaphoreType.DMA,
                       pltpu.SemaphoreType.DMA,
                       pltpu.SemaphoreType.DMA,
                       pltpu.VMEM_SHARED((NP, D), _f32)],
    )
    return k(table, rows, cols)


def _scale_body(x_ref, deg_ref, xs_ref, dinv_ref):
    x = x_ref[...]
    deg = jnp.max(deg_ref[...], axis=2, keepdims=True) + 1.0
    dinv = lax.rsqrt(deg)
    xs_ref[...] = dinv * x[None, :, :]
    dinv_ref[...] = jnp.broadcast_to(dinv, (2, NP, D))


def _scale_call(x_pad, deg16):
    return pl.pallas_call(
        _scale_body,
        out_shape=[jax.ShapeDtypeStruct((2, NP, D), _f32),
                   jax.ShapeDtypeStruct((2, NP, D), _f32)],
    )(x_pad, deg16)


def _elu(v):
    return jnp.where(v > 0, v, jnp.exp(jnp.minimum(v, 0.0)) - 1.0)


def _mlp_body(s_ref, dinv_ref, W1r, b1r, W2r, W3r, b3r, W4r, u_ref):
    for i, (Wa, ba, Wb) in ((0, (W1r, b1r, W2r)), (1, (W3r, b3r, W4r))):
        d = dinv_ref[i]
        a1 = d * s_ref[i]
        h = _elu(jnp.dot(a1, Wa[...], preferred_element_type=_f32) + ba[...])
        u_ref[i] = d * jnp.dot(h, Wb[...], preferred_element_type=_f32)


def _mlp_call(s1, dinv, W1, b1, W2, W3, b3, W4):
    RB = 1280
    grid = NP // RB
    blk3 = pl.BlockSpec((2, RB, D), lambda i: (0, i, 0))
    return pl.pallas_call(
        _mlp_body,
        grid=(grid,),
        in_specs=[blk3, blk3,
                  pl.BlockSpec((128, 256), lambda i: (0, 0)),
                  pl.BlockSpec((256,), lambda i: (0,)),
                  pl.BlockSpec((256, 128), lambda i: (0, 0)),
                  pl.BlockSpec((128, 256), lambda i: (0, 0)),
                  pl.BlockSpec((256,), lambda i: (0,)),
                  pl.BlockSpec((256, 128), lambda i: (0, 0))],
        out_specs=blk3,
        out_shape=jax.ShapeDtypeStruct((2, NP, D), _f32),
    )(s1, dinv, W1, b1, W2, W3, b3, W4)


def _final_body(s2_ref, dinv_ref, b2r, b4r, batch_ref, Wfc_ref, bfc_ref,
                out_ref):
    h2_td = _elu(dinv_ref[0] * s2_ref[0] + b2r[...])
    h2_bu = _elu(dinv_ref[1] * s2_ref[1] + b4r[...])
    seg = lax.broadcasted_iota(jnp.int32, (NG, NP), 0)
    bm = (batch_ref[...][None, :] == seg).astype(_f32)
    cnt = jnp.sum(bm, axis=1, keepdims=True)
    inv = 1.0 / jnp.maximum(cnt, 1.0)
    p_td = jnp.dot(bm, h2_td, preferred_element_type=_f32) * inv
    p_bu = jnp.dot(bm, h2_bu, preferred_element_type=_f32) * inv
    logits = (jnp.dot(p_td, Wfc_ref[0:D, :], preferred_element_type=_f32)
              + jnp.dot(p_bu, Wfc_ref[D:2 * D, :], preferred_element_type=_f32)
              + bfc_ref[...])
    m = jnp.max(logits, axis=1, keepdims=True)
    lse = jnp.log(jnp.sum(jnp.exp(logits - m), axis=1, keepdims=True)) + m
    out_ref[...] = logits - lse


def _final_call(s2, dinv, b2, b4, batch_pad, Wfc, bfc):
    return pl.pallas_call(
        _final_body,
        out_shape=jax.ShapeDtypeStruct((NG, 4), _f32),
    )(s2, dinv, b2, b4, batch_pad, Wfc, bfc)


def kernel(x, edge_index, BU_edge_index, batch,
           W1, b1, W2, b2, W3, b3, W4, b4, Wfc, bfc):
    i32 = jnp.int32
    pad_e = NEP - NE
    row_td = edge_index[0].astype(i32)
    col_td = edge_index[1].astype(i32)
    row_bu = BU_edge_index[0].astype(i32)
    col_bu = BU_edge_index[1].astype(i32)

    rows = jnp.stack([
        jnp.concatenate([row_td, jnp.full((pad_e,), NN, i32)]),
        jnp.concatenate([row_bu + NP, jnp.full((pad_e,), NP + NN, i32)]),
    ]).reshape(2 * TPC, CPT, CHUNK)
    cols = jnp.stack([
        jnp.concatenate([col_td, jnp.full((pad_e,), PAD_COL, i32)]),
        jnp.concatenate([col_bu, jnp.full((pad_e,), PAD_COL, i32)]),
    ]).reshape(2 * TPC, CPT, CHUNK)

    x_pad = jnp.concatenate([x, jnp.zeros((NP - NN, D), _f32)])
    batch_pad = jnp.concatenate(
        [batch.astype(i32), jnp.full((NP - NN,), NG, i32)])

    deg16 = _deg_call(cols)
    xs, dinv = _scale_call(x_pad, deg16)
    s1 = _agg_call(xs.reshape(2 * NP, D), rows, cols)
    u = _mlp_call(s1, dinv, W1, b1, W2, W3, b3, W4)
    s2 = _agg_call(u.reshape(2 * NP, D), rows, cols)
    return _final_call(s2, dinv, b2, b4, batch_pad, Wfc, bfc)


# speedup vs baseline: 1.3048x; 1.3048x over previous
import dataclasses
import functools

import jax
import jax.numpy as jnp
from jax import lax
from jax.experimental import pallas as pl
from jax.experimental.pallas import tpu as pltpu
from jax.experimental.pallas import tpu_sc as plsc

NN = 10000
NP = 10240
NE = 320000
CHUNK = 128
TPC = 16
CPT = 160
NEP = TPC * CPT * CHUNK
RPT = NP // TPC
PAD_COL = NN + 8
NG = 64
D = 128

_f32 = jnp.float32
_mesh = plsc.VectorSubcoreMesh(core_axis_name="c", subcore_axis_name="s")


IB = 40


def _deg_kernel(cols_hbm, out_hbm, col_v, hist):
    c = lax.axis_index("c")
    s = lax.axis_index("s")
    w = c * TPC + s

    @pl.loop(0, NP // 16)
    def _(i):
        hist[pl.ds(i * 16, 16)] = jnp.zeros((16,), _f32)

    ones16 = jnp.ones((16,), _f32)

    @pl.loop(0, CPT // IB)
    def _(blk):
        pltpu.sync_copy(cols_hbm.at[w, pl.ds(blk * IB, IB)], col_v)

        @pl.loop(0, IB)
        def _(g):
            @pl.loop(0, CHUNK // 16)
            def _(t):
                idx = col_v[g, pl.ds(t * 16, 16)]
                plsc.addupdate_scatter(hist, [idx], ones16)

    pltpu.sync_copy(hist, out_hbm.at[c, s])


def _deg_call(cols):
    cp = pltpu.CompilerParams()
    if "needs_layout_passes" in pltpu.CompilerParams.__dataclass_fields__:
        cp = dataclasses.replace(cp, needs_layout_passes=False)
    k = pl.kernel(
        _deg_kernel,
        out_type=jax.ShapeDtypeStruct((2, TPC, NP), _f32),
        mesh=_mesh,
        scratch_types=[pltpu.VMEM((IB, CHUNK), jnp.int32),
                       pltpu.VMEM((NP,), _f32)],
        compiler_params=cp,
    )
    return k(cols)


IB = 40


def _agg_kernel(table_hbm, rows_hbm, cols_hbm, out_hbm, row_v, col_v, gb0,
                gb1, sem0, sem1, acc):
    c = lax.axis_index("c")
    s = lax.axis_index("s")
    w = c * TPC + s

    pltpu.sync_copy(table_hbm.at[pl.ds(c * NP + s * RPT, RPT)],
                    acc.at[pl.ds(s * RPT, RPT)])
    plsc.subcore_barrier()

    @pl.loop(0, CPT // IB)
    def _(blk):
        pltpu.sync_copy(rows_hbm.at[w, pl.ds(blk * IB, IB)], row_v)
        pltpu.sync_copy(cols_hbm.at[w, pl.ds(blk * IB, IB)], col_v)
        pltpu.async_copy(table_hbm.at[row_v.at[0]], gb0, sem0)

        @pl.loop(0, IB, step=2)
        def _(g):
            pltpu.make_async_copy(table_hbm.at[row_v.at[g]], gb0, sem0).wait()
            pltpu.async_copy(table_hbm.at[row_v.at[g + 1]], gb1, sem1)
            pltpu.sync_copy(gb0, acc.at[col_v.at[g]], add=True)
            pltpu.make_async_copy(table_hbm.at[row_v.at[g + 1]], gb1,
                                  sem1).wait()

            @pl.when(g + 2 < IB)
            def _():
                pltpu.async_copy(table_hbm.at[row_v.at[g + 2]], gb0, sem0)

            pltpu.sync_copy(gb1, acc.at[col_v.at[g + 1]], add=True)

    plsc.subcore_barrier()
    pltpu.sync_copy(acc.at[pl.ds(s * RPT, RPT)],
                    out_hbm.at[c, pl.ds(s * RPT, RPT)])


def _agg_call(table, rows, cols):
    k = pl.kernel(
        _agg_kernel,
        out_type=jax.ShapeDtypeStruct((2, NP, D), _f32),
        mesh=_mesh,
        scratch_types=[pltpu.VMEM((IB, CHUNK), jnp.int32),
                       pltpu.VMEM((IB, CHUNK), jnp.int32),
                       pltpu.VMEM((CHUNK, D), _f32),
                       pltpu.VMEM((CHUNK, D), _f32),
                       pltpu.SemaphoreType.DMA,
                       pltpu.SemaphoreType.DMA,
                       pltpu.VMEM_SHARED((NP, D), _f32)],
    )
    return k(table, rows, cols)


def _scale_body(x_ref, deg_ref, xs_ref, dinv_ref):
    x = x_ref[...]
    deg = jnp.sum(deg_ref[...], axis=1)[:, :, None] + 1.0
    dinv = lax.rsqrt(deg)
    xs_ref[...] = dinv * x[None, :, :]
    dinv_ref[...] = jnp.broadcast_to(dinv, (2, NP, D))


def _scale_call(x_pad, deg16):
    return pl.pallas_call(
        _scale_body,
        out_shape=[jax.ShapeDtypeStruct((2, NP, D), _f32),
                   jax.ShapeDtypeStruct((2, NP, D), _f32)],
    )(x_pad, deg16)


def _elu(v):
    return jnp.where(v > 0, v, jnp.exp(jnp.minimum(v, 0.0)) - 1.0)


def _mlp_body(s_ref, dinv_ref, W1r, b1r, W2r, W3r, b3r, W4r, u_ref):
    for i, (Wa, ba, Wb) in ((0, (W1r, b1r, W2r)), (1, (W3r, b3r, W4r))):
        d = dinv_ref[i]
        a1 = d * s_ref[i]
        h = _elu(jnp.dot(a1, Wa[...], preferred_element_type=_f32) + ba[...])
        u_ref[i] = d * jnp.dot(h, Wb[...], preferred_element_type=_f32)


def _mlp_call(s1, dinv, W1, b1, W2, W3, b3, W4):
    RB = 1280
    grid = NP // RB
    blk3 = pl.BlockSpec((2, RB, D), lambda i: (0, i, 0))
    return pl.pallas_call(
        _mlp_body,
        grid=(grid,),
        in_specs=[blk3, blk3,
                  pl.BlockSpec((128, 256), lambda i: (0, 0)),
                  pl.BlockSpec((256,), lambda i: (0,)),
                  pl.BlockSpec((256, 128), lambda i: (0, 0)),
                  pl.BlockSpec((128, 256), lambda i: (0, 0)),
                  pl.BlockSpec((256,), lambda i: (0,)),
                  pl.BlockSpec((256, 128), lambda i: (0, 0))],
        out_specs=blk3,
        out_shape=jax.ShapeDtypeStruct((2, NP, D), _f32),
    )(s1, dinv, W1, b1, W2, W3, b3, W4)


def _final_body(s2_ref, dinv_ref, b2r, b4r, batch_ref, Wfc_ref, bfc_ref,
                out_ref):
    h2_td = _elu(dinv_ref[0] * s2_ref[0] + b2r[...])
    h2_bu = _elu(dinv_ref[1] * s2_ref[1] + b4r[...])
    seg = lax.broadcasted_iota(jnp.int32, (NG, NP), 0)
    bm = (batch_ref[...][None, :] == seg).astype(_f32)
    cnt = jnp.sum(bm, axis=1, keepdims=True)
    inv = 1.0 / jnp.maximum(cnt, 1.0)
    p_td = jnp.dot(bm, h2_td, preferred_element_type=_f32) * inv
    p_bu = jnp.dot(bm, h2_bu, preferred_element_type=_f32) * inv
    logits = (jnp.dot(p_td, Wfc_ref[0:D, :], preferred_element_type=_f32)
              + jnp.dot(p_bu, Wfc_ref[D:2 * D, :], preferred_element_type=_f32)
              + bfc_ref[...])
    m = jnp.max(logits, axis=1, keepdims=True)
    lse = jnp.log(jnp.sum(jnp.exp(logits - m), axis=1, keepdims=True)) + m
    out_ref[...] = logits - lse


def _final_call(s2, dinv, b2, b4, batch_pad, Wfc, bfc):
    return pl.pallas_call(
        _final_body,
        out_shape=jax.ShapeDtypeStruct((NG, 4), _f32),
    )(s2, dinv, b2, b4, batch_pad, Wfc, bfc)


def kernel(x, edge_index, BU_edge_index, batch,
           W1, b1, W2, b2, W3, b3, W4, b4, Wfc, bfc):
    i32 = jnp.int32
    pad_e = NEP - NE
    row_td = edge_index[0].astype(i32)
    col_td = edge_index[1].astype(i32)
    row_bu = BU_edge_index[0].astype(i32)
    col_bu = BU_edge_index[1].astype(i32)

    rows = jnp.stack([
        jnp.concatenate([row_td, jnp.full((pad_e,), NN, i32)]),
        jnp.concatenate([row_bu + NP, jnp.full((pad_e,), NP + NN, i32)]),
    ]).reshape(2 * TPC, CPT, CHUNK)
    cols = jnp.stack([
        jnp.concatenate([col_td, jnp.full((pad_e,), PAD_COL, i32)]),
        jnp.concatenate([col_bu, jnp.full((pad_e,), PAD_COL, i32)]),
    ]).reshape(2 * TPC, CPT, CHUNK)

    x_pad = jnp.concatenate([x, jnp.zeros((NP - NN, D), _f32)])
    batch_pad = jnp.concatenate(
        [batch.astype(i32), jnp.full((NP - NN,), NG, i32)])

    deg16 = _deg_call(cols)
    xs, dinv = _scale_call(x_pad, deg16)
    s1 = _agg_call(xs.reshape(2 * NP, D), rows, cols)
    u = _mlp_call(s1, dinv, W1, b1, W2, W3, b3, W4)
    s2 = _agg_call(u.reshape(2 * NP, D), rows, cols)
    return _final_call(s2, dinv, b2, b4, batch_pad, Wfc, bfc)


# speedup vs baseline: 1.3063x; 1.0011x over previous
import dataclasses
import functools

import jax
import jax.numpy as jnp
from jax import lax
from jax.experimental import pallas as pl
from jax.experimental.pallas import tpu as pltpu
from jax.experimental.pallas import tpu_sc as plsc

NN = 10000
NP = 10240
NE = 320000
CHUNK = 128
TPC = 16
CPT = 160
NEP = TPC * CPT * CHUNK
RPT = NP // TPC
PAD_COL = NN + 8
NG = 64
D = 128

_f32 = jnp.float32
_mesh = plsc.VectorSubcoreMesh(core_axis_name="c", subcore_axis_name="s")


IB = 40


def _deg_kernel(cols_hbm, out_hbm, col_v, hist):
    c = lax.axis_index("c")
    s = lax.axis_index("s")
    w = c * TPC + s

    @pl.loop(0, NP // 16)
    def _(i):
        hist[pl.ds(i * 16, 16)] = jnp.zeros((16,), _f32)

    ones16 = jnp.ones((16,), _f32)

    @pl.loop(0, CPT // IB)
    def _(blk):
        pltpu.sync_copy(cols_hbm.at[w, pl.ds(blk * IB, IB)], col_v)

        @pl.loop(0, IB)
        def _(g):
            @pl.loop(0, CHUNK // 16)
            def _(t):
                idx = col_v[g, pl.ds(t * 16, 16)]
                plsc.addupdate_scatter(hist, [idx], ones16)

    pltpu.sync_copy(hist, out_hbm.at[c, s])


def _deg_call(cols):
    cp = pltpu.CompilerParams()
    if "needs_layout_passes" in pltpu.CompilerParams.__dataclass_fields__:
        cp = dataclasses.replace(cp, needs_layout_passes=False)
    k = pl.kernel(
        _deg_kernel,
        out_type=jax.ShapeDtypeStruct((2, TPC, NP), _f32),
        mesh=_mesh,
        scratch_types=[pltpu.VMEM((IB, CHUNK), jnp.int32),
                       pltpu.VMEM((NP,), _f32)],
        compiler_params=cp,
    )
    return k(cols)


def _agg_kernel(table_hbm, rows_hbm, cols_hbm, out_hbm, row_v, col_v, gb0,
                gb1, sem0, sem1, acc):
    c = lax.axis_index("c")
    s = lax.axis_index("s")
    w = c * TPC + s

    pltpu.sync_copy(table_hbm.at[pl.ds(c * NP + s * RPT, RPT)],
                    acc.at[pl.ds(s * RPT, RPT)])
    plsc.subcore_barrier()

    @pl.loop(0, CPT // IB)
    def _(blk):
        pltpu.sync_copy(rows_hbm.at[w, pl.ds(blk * IB, IB)], row_v)
        pltpu.sync_copy(cols_hbm.at[w, pl.ds(blk * IB, IB)], col_v)
        pltpu.async_copy(table_hbm.at[row_v.at[0]], gb0, sem0)

        @pl.loop(0, IB, step=2)
        def _(g):
            pltpu.make_async_copy(table_hbm.at[row_v.at[g]], gb0, sem0).wait()
            pltpu.async_copy(table_hbm.at[row_v.at[g + 1]], gb1, sem1)
            pltpu.sync_copy(gb0, acc.at[col_v.at[g]], add=True)
            pltpu.make_async_copy(table_hbm.at[row_v.at[g + 1]], gb1,
                                  sem1).wait()

            @pl.when(g + 2 < IB)
            def _():
                pltpu.async_copy(table_hbm.at[row_v.at[g + 2]], gb0, sem0)

            pltpu.sync_copy(gb1, acc.at[col_v.at[g + 1]], add=True)

    plsc.subcore_barrier()
    pltpu.sync_copy(acc.at[pl.ds(s * RPT, RPT)],
                    out_hbm.at[c, pl.ds(s * RPT, RPT)])


def _agg_call(table, rows, cols):
    k = pl.kernel(
        _agg_kernel,
        out_type=jax.ShapeDtypeStruct((2, NP, D), _f32),
        mesh=_mesh,
        scratch_types=[pltpu.VMEM((IB, CHUNK), jnp.int32),
                       pltpu.VMEM((IB, CHUNK), jnp.int32),
                       pltpu.VMEM((CHUNK, D), _f32),
                       pltpu.VMEM((CHUNK, D), _f32),
                       pltpu.SemaphoreType.DMA,
                       pltpu.SemaphoreType.DMA,
                       pltpu.VMEM_SHARED((NP, D), _f32)],
    )
    return k(table, rows, cols)


def _scale_body(x_ref, deg_ref, xs_ref, dinv_ref):
    x = x_ref[...]
    deg = jnp.sum(deg_ref[...], axis=1)[:, :, None] + 1.0
    dinv = lax.rsqrt(deg)
    xs_ref[...] = dinv * x[None, :, :]
    dinv_ref[...] = jnp.broadcast_to(dinv, (2, NP, D))


def _scale_call(x_pad, deg16):
    return pl.pallas_call(
        _scale_body,
        out_shape=[jax.ShapeDtypeStruct((2, NP, D), _f32),
                   jax.ShapeDtypeStruct((2, NP, D), _f32)],
    )(x_pad, deg16)


def _elu(v):
    return jnp.where(v > 0, v, jnp.exp(jnp.minimum(v, 0.0)) - 1.0)


def _mlp_body(s_ref, dinv_ref, W1r, b1r, W2r, W3r, b3r, W4r, u_ref):
    for i, (Wa, ba, Wb) in ((0, (W1r, b1r, W2r)), (1, (W3r, b3r, W4r))):
        d = dinv_ref[i]
        a1 = d * s_ref[i]
        h = _elu(jnp.dot(a1, Wa[...], preferred_element_type=_f32) + ba[...])
        u_ref[i] = d * jnp.dot(h, Wb[...], preferred_element_type=_f32)


def _mlp_call(s1, dinv, W1, b1, W2, W3, b3, W4):
    RB = 1280
    grid = NP // RB
    blk3 = pl.BlockSpec((2, RB, D), lambda i: (0, i, 0))
    return pl.pallas_call(
        _mlp_body,
        grid=(grid,),
        in_specs=[blk3, blk3,
                  pl.BlockSpec((128, 256), lambda i: (0, 0)),
                  pl.BlockSpec((256,), lambda i: (0,)),
                  pl.BlockSpec((256, 128), lambda i: (0, 0)),
                  pl.BlockSpec((128, 256), lambda i: (0, 0)),
                  pl.BlockSpec((256,), lambda i: (0,)),
                  pl.BlockSpec((256, 128), lambda i: (0, 0))],
        out_specs=blk3,
        out_shape=jax.ShapeDtypeStruct((2, NP, D), _f32),
    )(s1, dinv, W1, b1, W2, W3, b3, W4)


def _final_body(s2_ref, dinv_ref, b2r, b4r, batch_ref, Wfc_ref, bfc_ref,
                out_ref):
    h2_td = _elu(dinv_ref[0] * s2_ref[0] + b2r[...])
    h2_bu = _elu(dinv_ref[1] * s2_ref[1] + b4r[...])
    seg = lax.broadcasted_iota(jnp.int32, (NG, NP), 0)
    bm = (batch_ref[...][None, :] == seg).astype(_f32)
    cnt = jnp.sum(bm, axis=1, keepdims=True)
    inv = 1.0 / jnp.maximum(cnt, 1.0)
    p_td = jnp.dot(bm, h2_td, preferred_element_type=_f32) * inv
    p_bu = jnp.dot(bm, h2_bu, preferred_element_type=_f32) * inv
    logits = (jnp.dot(p_td, Wfc_ref[0:D, :], preferred_element_type=_f32)
              + jnp.dot(p_bu, Wfc_ref[D:2 * D, :], preferred_element_type=_f32)
              + bfc_ref[...])
    m = jnp.max(logits, axis=1, keepdims=True)
    lse = jnp.log(jnp.sum(jnp.exp(logits - m), axis=1, keepdims=True)) + m
    out_ref[...] = logits - lse


def _final_call(s2, dinv, b2, b4, batch_pad, Wfc, bfc):
    return pl.pallas_call(
        _final_body,
        out_shape=jax.ShapeDtypeStruct((NG, 4), _f32),
    )(s2, dinv, b2, b4, batch_pad, Wfc, bfc)


def kernel(x, edge_index, BU_edge_index, batch,
           W1, b1, W2, b2, W3, b3, W4, b4, Wfc, bfc):
    i32 = jnp.int32
    pad_e = NEP - NE
    row_td = edge_index[0].astype(i32)
    col_td = edge_index[1].astype(i32)
    row_bu = BU_edge_index[0].astype(i32)
    col_bu = BU_edge_index[1].astype(i32)

    rows = jnp.stack([
        jnp.concatenate([row_td, jnp.full((pad_e,), NN, i32)]),
        jnp.concatenate([row_bu + NP, jnp.full((pad_e,), NP + NN, i32)]),
    ]).reshape(2 * TPC, CPT, CHUNK)
    cols = jnp.stack([
        jnp.concatenate([col_td, jnp.full((pad_e,), PAD_COL, i32)]),
        jnp.concatenate([col_bu, jnp.full((pad_e,), PAD_COL, i32)]),
    ]).reshape(2 * TPC, CPT, CHUNK)

    x_pad = jnp.concatenate([x, jnp.zeros((NP - NN, D), _f32)])
    batch_pad = jnp.concatenate(
        [batch.astype(i32), jnp.full((NP - NN,), NG, i32)])

    deg16 = _deg_call(cols)
    xs, dinv = _scale_call(x_pad, deg16)
    s1 = _agg_call(xs.reshape(2 * NP, D), rows, cols)
    u = _mlp_call(s1, dinv, W1, b1, W2, W3, b3, W4)
    s2 = _agg_call(u.reshape(2 * NP, D), rows, cols)
    return _final_call(s2, dinv, b2, b4, batch_pad, Wfc, bfc)
